# baseline (speedup 1.0000x reference)
import jax
import jax.numpy as jnp
from jax import lax
from jax.experimental import pallas as pl
from jax.experimental.pallas import tpu as pltpu
from jax.experimental.pallas import tpu_sc as plsc

N_NODES = 10000
N_EDGES = 320000
D_FEAT = 128

NC = 2
NS = 16
NW = NC * NS

CHUNK = 128
ROWS_PER_W = 80
N_ROWS = NW * ROWS_PER_W
E_PAD = N_ROWS * CHUNK
ZROWS = 632
N_ACC = NS * ZROWS


def _sc_accumulate(x_hbm, src_hbm, dst_hbm, part_hbm,
                   acc_sh, src_v, dst_v, rows_v, gsem, isem):
    c = lax.axis_index("c")
    s = lax.axis_index("s")
    wid = s * NC + c

    def _zero_vmem(i, _):
        for j in range(8):
            rows_v[0, i, pl.ds(j * 16, 16)] = jnp.zeros((16,), jnp.float32)
        return 0
    lax.fori_loop(0, CHUNK, _zero_vmem, 0)
    zbase = s * ZROWS
    for k in range(4):
        pltpu.sync_copy(rows_v.at[0],
                        acc_sh.at[pl.ds(zbase + k * CHUNK, CHUNK), :])
    pltpu.sync_copy(rows_v.at[0, pl.ds(0, ZROWS - 4 * CHUNK), :],
                    acc_sh.at[pl.ds(zbase + 4 * CHUNK, ZROWS - 4 * CHUNK), :])
    plsc.subcore_barrier()

    rbase = wid * ROWS_PER_W
    NQ = 4

    def _gather_start(b, q):
        pltpu.async_copy(x_hbm.at[src_v.at[q]], rows_v.at[b], gsem.at[b])

    def _gather_wait(b, q):
        pltpu.make_async_copy(x_hbm.at[src_v.at[q]], rows_v.at[b],
                              gsem.at[b]).wait()

    def _idx_fetch(q, row):
        pltpu.async_copy(src_hbm.at[row, :], src_v.at[q], isem.at[q])
        pltpu.async_copy(dst_hbm.at[row, :], dst_v.at[q], isem.at[q])

    def _idx_wait(q, row):
        pltpu.make_async_copy(src_hbm.at[row, :], src_v.at[q],
                              isem.at[q]).wait()
        pltpu.make_async_copy(dst_hbm.at[row, :], dst_v.at[q],
                              isem.at[q]).wait()

    pltpu.sync_copy(src_hbm.at[rbase, :], src_v.at[0])
    pltpu.sync_copy(dst_hbm.at[rbase, :], dst_v.at[0])
    _gather_start(0, 0)
    _idx_fetch(1, rbase + 1)
    _idx_fetch(2, rbase + 2)

    NP = ROWS_PER_W // NQ

    def _macro(p, _):
        for q in range(NQ):
            g4 = NQ * p + q
            b = q % 2
            b2 = 1 - b
            q1 = (q + 1) % NQ
            q3 = (q + 3) % NQ
            _gather_wait(b, q)
            is_last = (q == NQ - 1)

            def _advance(_b2=b2, _q=q, _q1=q1, _q3=q3, _g=g4):
                _idx_wait(_q1, rbase + _g + 1)
                _gather_start(_b2, _q1)
                if _q == 0:
                    _idx_fetch(_q3, rbase + _g + 3)
                else:
                    @pl.when(p < NP - 1)
                    def _():
                        _idx_fetch(_q3, rbase + _g + 3)

            if is_last:
                @pl.when(p < NP - 1)
                def _():
                    _advance()
            else:
                _advance()
            pltpu.sync_copy(rows_v.at[b], acc_sh.at[dst_v.at[q]], add=True)
        return 0
    lax.fori_loop(0, NP, _macro, 0)
    plsc.subcore_barrier()

    wbase = s * ZROWS
    pltpu.sync_copy(acc_sh.at[pl.ds(wbase, ZROWS), :],
                    part_hbm.at[c, pl.ds(wbase, ZROWS), :])


def _combine_body(p_ref, o_ref):
    o_ref[...] = p_ref[0] + p_ref[1]


@jax.jit
def kernel(x, edge_index):
    src = jnp.concatenate(
        [edge_index[0], jnp.zeros((E_PAD - N_EDGES,), jnp.int32)])
    dst = jnp.concatenate(
        [edge_index[1], jnp.full((E_PAD - N_EDGES,), N_NODES, jnp.int32)])
    src2d = src.reshape(N_ROWS, CHUNK)
    dst2d = dst.reshape(N_ROWS, CHUNK)

    mesh = plsc.VectorSubcoreMesh(core_axis_name="c", subcore_axis_name="s",
                                  num_cores=NC, num_subcores=NS)
    partials = pl.kernel(
        _sc_accumulate,
        out_type=jax.ShapeDtypeStruct((NC, N_ACC, D_FEAT), jnp.float32),
        mesh=mesh,
        scratch_types=[
            pltpu.VMEM_SHARED((N_ACC, D_FEAT), jnp.float32),
            pltpu.VMEM((4, CHUNK), jnp.int32),
            pltpu.VMEM((4, CHUNK), jnp.int32),
            pltpu.VMEM((2, CHUNK, D_FEAT), jnp.float32),
            pltpu.SemaphoreType.DMA((2,)),
            pltpu.SemaphoreType.DMA((4,)),
        ],
    )(x, src2d, dst2d)

    out = pl.pallas_call(
        _combine_body,
        out_shape=jax.ShapeDtypeStruct((N_NODES, D_FEAT), jnp.float32),
        grid=(10,),
        in_specs=[pl.BlockSpec((NC, N_NODES // 10, D_FEAT),
                               lambda i: (0, i, 0))],
        out_specs=pl.BlockSpec((N_NODES // 10, D_FEAT), lambda i: (i, 0)),
    )(partials)
    return out
